# baseline (speedup 1.0000x reference)
import functools

import jax
import jax.numpy as jnp
from jax import lax
from jax.experimental import pallas as pl
from jax.experimental.pallas import tpu as pltpu
from jax.experimental.pallas import tpu_sc as plsc

N = 10000
E = 320000
F_IN = 128
HID = 64
C = 64
W = 72
R = 1000
NT = 32
NP = 10240

CHA = 128
NCA = E // CHA
NJA_LO = NCA // NT
NJA_HI = NJA_LO + 1
NRA = NCA % NT
SBA = 80

CHB = 64
NCB = E // CHB
NJB_LO = NCB // NT
NJB_HI = NJB_LO + 1
NRB = NCB % NT
SBB = 160
NTAB = N

_f32 = jnp.float32
_i32 = jnp.int32


def _k1_body(x_ref, wl_ref, wr_ref, xlaug_ref, xr_ref):
    xb = x_ref[...]
    dn = (((1,), (1,)), ((), ()))
    xl = lax.dot_general(xb, wl_ref[...], dn, preferred_element_type=_f32)
    xr = lax.dot_general(xb, wr_ref[...], dn, preferred_element_type=_f32)
    pad = jnp.concatenate([xl, jnp.zeros((R, W - HID), _f32)], axis=1)
    col = lax.broadcasted_iota(_i32, (R, W), 1)
    xlaug_ref[...] = jnp.where(col == HID, 1.0, pad)
    xr_ref[...] = xr


_k1 = pl.pallas_call(
    _k1_body,
    grid=(N // R,),
    in_specs=[
        pl.BlockSpec((R, F_IN), lambda i: (i, 0)),
        pl.BlockSpec((HID, F_IN), lambda i: (0, 0)),
        pl.BlockSpec((HID, F_IN), lambda i: (0, 0)),
    ],
    out_specs=[
        pl.BlockSpec((R, W), lambda i: (i, 0)),
        pl.BlockSpec((R, HID), lambda i: (i, 0)),
    ],
    out_shape=[
        jax.ShapeDtypeStruct((N, W), _f32),
        jax.ShapeDtypeStruct((N, HID), _f32),
    ],
)


_mesh = plsc.VectorSubcoreMesh(
    core_axis_name="c", subcore_axis_name="s", num_cores=2, num_subcores=16
)
_RPT_A = NP // 16


@functools.partial(
    pl.kernel,
    out_type=jax.ShapeDtypeStruct((2, NP, W), _f32),
    mesh=_mesh,
    scratch_types=[
        pltpu.VMEM_SHARED((NP, W), _f32),
        tuple(pltpu.VMEM((CHA,), _i32) for _ in range(4)),
        tuple(pltpu.VMEM((CHA,), _i32) for _ in range(4)),
        tuple(pltpu.VMEM((CHA, W), _f32) for _ in range(2)),
        tuple(pltpu.SemaphoreType.DMA for _ in range(2)),
        pltpu.SemaphoreType.DMA,
        tuple(pltpu.SemaphoreType.DMA for _ in range(2)),
    ],
    compiler_params=pltpu.CompilerParams(use_tc_tiling_on_sc=False, needs_layout_passes=False),
)
def _passA(xlaug_hbm, src_hbm, dst_hbm, zeros_hbm, out_hbm,
           acc, sbufs, dbufs, rowss, semi, semg, sems):
    cid = lax.axis_index("c")
    sid = lax.axis_index("s")
    wid = sid * 2 + cid
    pltpu.sync_copy(zeros_hbm.at[pl.ds(0, _RPT_A)],
                    acc.at[pl.ds(sid * _RPT_A, _RPT_A)])
    plsc.subcore_barrier()

    def issue_idx(j, ib, pb):
        base = (wid + j * NT) * CHA
        pltpu.async_copy(src_hbm.at[pl.ds(base, CHA)], sbufs[ib], semi[pb])
        pltpu.async_copy(dst_hbm.at[pl.ds(base, CHA)], dbufs[ib], semi[pb])

    def wait_idx(ib, pb):
        pltpu.make_async_copy(src_hbm.at[pl.ds(0, CHA)], sbufs[ib],
                              semi[pb]).wait()
        pltpu.make_async_copy(dst_hbm.at[pl.ds(0, CHA)], dbufs[ib],
                              semi[pb]).wait()

    def wait_scatter(rb):
        pltpu.make_async_copy(rowss[rb], acc.at[dbufs[0]], sems[rb]).wait()

    nj = jnp.where(wid < NRA, NJA_HI, NJA_LO)
    issue_idx(0, 0, 0)

    def outer(i, carry):
        jj = i * 4
        for b in range(4):
            j = jj + b
            ib, pb, rb = b, b % 2, b % 2

            @pl.when(j < nj)
            def _():
                @pl.when(j + 1 < nj)
                def _():
                    issue_idx(j + 1, (b + 1) % 4, (b + 1) % 2)
                wait_idx(ib, pb)

                @pl.when(j >= 2)
                def _():
                    wait_scatter(rb)
                pltpu.async_copy(xlaug_hbm.at[sbufs[ib]], rowss[rb],
                                 semg).wait()
                pltpu.async_copy(rowss[rb], acc.at[dbufs[ib]], sems[rb],
                                 add=True)
        return carry

    lax.fori_loop(0, SBA // 4, outer, 0)
    wait_scatter(0)
    wait_scatter(1)
    plsc.subcore_barrier()
    pltpu.sync_copy(acc.at[pl.ds(sid * _RPT_A, _RPT_A)],
                    out_hbm.at[cid, pl.ds(sid * _RPT_A, _RPT_A)])


def _k2_body(accA_ref, xr_ref, b1_ref, w2_ref, asrc_ref, adst_ref,
             u_ref, s_ref, d_ref, z_ref):
    acc = accA_ref[0] + accA_ref[1]
    deg = jnp.maximum(acc[:, HID], 1.0)
    agg = acc[:, :HID] / deg[:, None]
    h = jnp.maximum(agg + xr_ref[...] + b1_ref[...][None, :], 0.0)
    dn = (((1,), (1,)), ((), ()))
    z = lax.dot_general(h, w2_ref[...], dn, preferred_element_type=_f32)
    s = jnp.sum(z * asrc_ref[...][None, :], axis=1)
    d = jnp.sum(z * adst_ref[...][None, :], axis=1)
    zpad = jnp.concatenate([z, jnp.zeros((R, W - HID), _f32)], axis=1)
    col = lax.broadcasted_iota(_i32, (R, W), 1)
    zaug = jnp.where(col == HID, 1.0, zpad)
    u_ref[0] = jnp.exp(s)[:, None] * zaug
    u_ref[1] = jnp.exp(0.2 * s)[:, None] * zaug
    s_ref[...] = s[:, None]
    d_ref[...] = d[:, None]
    z_ref[...] = z


_k2 = pl.pallas_call(
    _k2_body,
    grid=(N // R,),
    in_specs=[
        pl.BlockSpec((2, R, W), lambda i: (0, i, 0)),
        pl.BlockSpec((R, HID), lambda i: (i, 0)),
        pl.BlockSpec((HID,), lambda i: (0,)),
        pl.BlockSpec((C, HID), lambda i: (0, 0)),
        pl.BlockSpec((C,), lambda i: (0,)),
        pl.BlockSpec((C,), lambda i: (0,)),
    ],
    out_specs=[
        pl.BlockSpec((2, R, W), lambda i: (0, i, 0)),
        pl.BlockSpec((R, 1), lambda i: (i, 0)),
        pl.BlockSpec((R, 1), lambda i: (i, 0)),
        pl.BlockSpec((R, C), lambda i: (i, 0)),
    ],
    out_shape=[
        jax.ShapeDtypeStruct((2, NP, W), _f32),
        jax.ShapeDtypeStruct((N, 1), _f32),
        jax.ShapeDtypeStruct((N, 1), _f32),
        jax.ShapeDtypeStruct((N, C), _f32),
    ],
)


_RPT_B = 2 * NP // 16


@functools.partial(
    pl.kernel,
    out_type=jax.ShapeDtypeStruct((2, 2 * NP, W), _f32),
    mesh=_mesh,
    scratch_types=[
        pltpu.VMEM_SHARED((2 * NP, W), _f32),
        pltpu.VMEM((NTAB,), _f32),
        pltpu.VMEM((NTAB,), _f32),
        tuple(pltpu.VMEM((CHB,), _i32) for _ in range(4)),
        tuple(pltpu.VMEM((CHB,), _i32) for _ in range(4)),
        tuple(pltpu.VMEM((CHB,), _i32) for _ in range(4)),
        tuple(pltpu.VMEM((CHB,), _i32) for _ in range(4)),
        tuple(pltpu.VMEM((CHB, W), _f32) for _ in range(2)),
        tuple(pltpu.SemaphoreType.DMA for _ in range(2)),
        pltpu.SemaphoreType.DMA,
        tuple(pltpu.SemaphoreType.DMA for _ in range(2)),
    ],
    compiler_params=pltpu.CompilerParams(use_tc_tiling_on_sc=False, needs_layout_passes=False),
)
def _passB(u_hbm, s_hbm, d_hbm, src_hbm, dst_hbm, zeros_hbm, out_hbm,
           acc, stab, dtab, sbufs, dbufs, gbufs, wbufs, rowss,
           semi, semg, sems):
    cid = lax.axis_index("c")
    sid = lax.axis_index("s")
    wid = sid * 2 + cid
    pltpu.sync_copy(zeros_hbm, acc.at[pl.ds(sid * _RPT_B, _RPT_B)])
    pltpu.sync_copy(s_hbm, stab)
    pltpu.sync_copy(d_hbm, dtab)
    plsc.subcore_barrier()

    def issue_idx(j, ib, pb):
        base = (wid + j * NT) * CHB
        pltpu.async_copy(src_hbm.at[pl.ds(base, CHB)], sbufs[ib], semi[pb])
        pltpu.async_copy(dst_hbm.at[pl.ds(base, CHB)], dbufs[ib], semi[pb])

    def wait_idx(ib, pb):
        pltpu.make_async_copy(src_hbm.at[pl.ds(0, CHB)], sbufs[ib],
                              semi[pb]).wait()
        pltpu.make_async_copy(dst_hbm.at[pl.ds(0, CHB)], dbufs[ib],
                              semi[pb]).wait()

    def wait_scatter(rb):
        pltpu.make_async_copy(rowss[rb], acc.at[wbufs[0]], sems[rb]).wait()

    nj = jnp.where(wid < NRB, NJB_HI, NJB_LO)
    issue_idx(0, 0, 0)

    def outer(i, carry):
        jj = i * 4
        for b in range(4):
            j = jj + b
            ib, pb, rb = b, b % 2, b % 2

            @pl.when(j < nj)
            def _():
                @pl.when(j + 1 < nj)
                def _():
                    issue_idx(j + 1, (b + 1) % 4, (b + 1) % 2)
                wait_idx(ib, pb)
                for q in range(CHB // 16):
                    sv = sbufs[ib][pl.ds(q * 16, 16)]
                    dv = dbufs[ib][pl.ds(q * 16, 16)]
                    s16 = plsc.load_gather(stab, [sv])
                    d16 = plsc.load_gather(dtab, [dv])
                    neg = (s16 + d16) <= 0.0
                    off = jnp.where(neg, jnp.full((16,), NP, _i32),
                                    jnp.zeros((16,), _i32))
                    gbufs[ib][pl.ds(q * 16, 16)] = sv + off
                    wbufs[ib][pl.ds(q * 16, 16)] = dv + off

                @pl.when(j >= 2)
                def _():
                    wait_scatter(rb)
                pltpu.async_copy(u_hbm.at[gbufs[ib]], rowss[rb],
                                 semg).wait()
                pltpu.async_copy(rowss[rb], acc.at[wbufs[ib]], sems[rb],
                                 add=True)
        return carry

    lax.fori_loop(0, SBB // 4, outer, 0)
    wait_scatter(0)
    wait_scatter(1)
    plsc.subcore_barrier()
    pltpu.sync_copy(acc.at[pl.ds(sid * _RPT_B, _RPT_B)],
                    out_hbm.at[cid, pl.ds(sid * _RPT_B, _RPT_B)])


def _k3_body(accB_ref, s_ref, d_ref, z_ref, b2_ref, out_ref):
    p = accB_ref[0, 0] + accB_ref[1, 0]
    q = accB_ref[0, 1] + accB_ref[1, 1]
    s = s_ref[...][:, 0]
    d = d_ref[...][:, 0]
    full = jnp.exp(d)[:, None] * p + jnp.exp(0.2 * d)[:, None] * q
    t = s + d
    w = jnp.where(t > 0.0, jnp.exp(t), jnp.exp(0.2 * t))
    numer = full[:, :C] + w[:, None] * z_ref[...]
    denom = full[:, HID] + w
    logits = numer / denom[:, None] + b2_ref[...][None, :]
    mx = jnp.max(logits, axis=1, keepdims=True)
    sh = logits - mx
    out_ref[...] = sh - jnp.log(jnp.sum(jnp.exp(sh), axis=1, keepdims=True))


_k3 = pl.pallas_call(
    _k3_body,
    grid=(N // R,),
    in_specs=[
        pl.BlockSpec((2, 2, R, W), lambda i: (0, 0, i, 0)),
        pl.BlockSpec((R, 1), lambda i: (i, 0)),
        pl.BlockSpec((R, 1), lambda i: (i, 0)),
        pl.BlockSpec((R, C), lambda i: (i, 0)),
        pl.BlockSpec((C,), lambda i: (0,)),
    ],
    out_specs=pl.BlockSpec((R, C), lambda i: (i, 0)),
    out_shape=jax.ShapeDtypeStruct((N, C), _f32),
)


def kernel(x, edge_index, Wl, Wr, b1, W2, a_src, a_dst, b2):
    zeros2 = jnp.zeros((_RPT_B, W), _f32)
    pad = 12032
    src = jnp.concatenate([edge_index[0], jnp.zeros((pad,), jnp.int32)])
    dst = jnp.concatenate(
        [edge_index[1], N + (jnp.arange(pad, dtype=jnp.int32) % (NP - N))])
    xlaug, xr = _k1(x, Wl, Wr)
    accA = _passA(xlaug, src, dst, zeros2)
    u, s, d, z = _k2(accA, xr, b1, W2, a_src, a_dst)
    accB = _passB(u.reshape(2 * NP, W), s.reshape(N), d.reshape(N),
                  src, dst, zeros2)
    return _k3(accB.reshape(2, 2, NP, W), s, d, z, b2)


# speedup vs baseline: 1.1306x; 1.1306x over previous
import functools

import jax
import jax.numpy as jnp
from jax import lax
from jax.experimental import pallas as pl
from jax.experimental.pallas import tpu as pltpu
from jax.experimental.pallas import tpu_sc as plsc

N = 10000
E = 320000
F_IN = 128
HID = 64
C = 64
W = 72
R = 1000
NT = 32
NP = 10240

CHA = 128
NCA = E // CHA
NJA_LO = NCA // NT
NJA_HI = NJA_LO + 1
NRA = NCA % NT
SBA = 80

CHB = 128
NJB_LO = NJA_LO
NJB_HI = NJA_HI
NRB = NRA
SBB = 80
NOFF = 10048
NTAB = NOFF

_f32 = jnp.float32
_i32 = jnp.int32


def _k1_body(x_ref, wl_ref, wr_ref, xlaug_ref, xr_ref):
    xb = x_ref[...]
    dn = (((1,), (1,)), ((), ()))
    xl = lax.dot_general(xb, wl_ref[...], dn, preferred_element_type=_f32)
    xr = lax.dot_general(xb, wr_ref[...], dn, preferred_element_type=_f32)
    pad = jnp.concatenate([xl, jnp.zeros((R, W - HID), _f32)], axis=1)
    col = lax.broadcasted_iota(_i32, (R, W), 1)
    xlaug_ref[...] = jnp.where(col == HID, 1.0, pad)
    xr_ref[...] = xr


_k1 = pl.pallas_call(
    _k1_body,
    grid=(N // R,),
    in_specs=[
        pl.BlockSpec((R, F_IN), lambda i: (i, 0)),
        pl.BlockSpec((HID, F_IN), lambda i: (0, 0)),
        pl.BlockSpec((HID, F_IN), lambda i: (0, 0)),
    ],
    out_specs=[
        pl.BlockSpec((R, W), lambda i: (i, 0)),
        pl.BlockSpec((R, HID), lambda i: (i, 0)),
    ],
    out_shape=[
        jax.ShapeDtypeStruct((N, W), _f32),
        jax.ShapeDtypeStruct((N, HID), _f32),
    ],
)


_mesh = plsc.VectorSubcoreMesh(
    core_axis_name="c", subcore_axis_name="s", num_cores=2, num_subcores=16
)
_RPT_A = NP // 16


@functools.partial(
    pl.kernel,
    out_type=jax.ShapeDtypeStruct((2, NP, W), _f32),
    mesh=_mesh,
    scratch_types=[
        pltpu.VMEM_SHARED((NP, W), _f32),
        tuple(pltpu.VMEM((CHA,), _i32) for _ in range(4)),
        tuple(pltpu.VMEM((CHA,), _i32) for _ in range(4)),
        tuple(pltpu.VMEM((CHA, W), _f32) for _ in range(2)),
        tuple(pltpu.SemaphoreType.DMA for _ in range(2)),
        pltpu.SemaphoreType.DMA,
        tuple(pltpu.SemaphoreType.DMA for _ in range(2)),
    ],
    compiler_params=pltpu.CompilerParams(use_tc_tiling_on_sc=False, needs_layout_passes=False),
)
def _passA(xlaug_hbm, src_hbm, dst_hbm, zeros_hbm, out_hbm,
           acc, sbufs, dbufs, rowss, semi, semg, sems):
    cid = lax.axis_index("c")
    sid = lax.axis_index("s")
    wid = sid * 2 + cid
    pltpu.sync_copy(zeros_hbm.at[pl.ds(0, _RPT_A)],
                    acc.at[pl.ds(sid * _RPT_A, _RPT_A)])
    plsc.subcore_barrier()

    def issue_idx(j, ib, pb):
        base = (wid + j * NT) * CHA
        pltpu.async_copy(src_hbm.at[pl.ds(base, CHA)], sbufs[ib], semi[pb])
        pltpu.async_copy(dst_hbm.at[pl.ds(base, CHA)], dbufs[ib], semi[pb])

    def wait_idx(ib, pb):
        pltpu.make_async_copy(src_hbm.at[pl.ds(0, CHA)], sbufs[ib],
                              semi[pb]).wait()
        pltpu.make_async_copy(dst_hbm.at[pl.ds(0, CHA)], dbufs[ib],
                              semi[pb]).wait()

    def wait_scatter(rb):
        pltpu.make_async_copy(rowss[rb], acc.at[dbufs[0]], sems[rb]).wait()

    nj = jnp.where(wid < NRA, NJA_HI, NJA_LO)
    issue_idx(0, 0, 0)

    def outer(i, carry):
        jj = i * 4
        for b in range(4):
            j = jj + b
            ib, pb, rb = b, b % 2, b % 2

            @pl.when(j < nj)
            def _():
                @pl.when(j + 1 < nj)
                def _():
                    issue_idx(j + 1, (b + 1) % 4, (b + 1) % 2)
                wait_idx(ib, pb)

                @pl.when(j >= 2)
                def _():
                    wait_scatter(rb)
                pltpu.async_copy(xlaug_hbm.at[sbufs[ib]], rowss[rb],
                                 semg).wait()
                pltpu.async_copy(rowss[rb], acc.at[dbufs[ib]], sems[rb],
                                 add=True)
        return carry

    lax.fori_loop(0, SBA // 4, outer, 0)
    wait_scatter(0)
    wait_scatter(1)
    plsc.subcore_barrier()
    pltpu.sync_copy(acc.at[pl.ds(sid * _RPT_A, _RPT_A)],
                    out_hbm.at[cid, pl.ds(sid * _RPT_A, _RPT_A)])


def _k2_body(accA_ref, xr_ref, b1_ref, w2_ref, asrc_ref, adst_ref,
             u_ref, s_ref, d_ref, z_ref):
    acc = accA_ref[0] + accA_ref[1]
    deg = jnp.maximum(acc[:, HID], 1.0)
    agg = acc[:, :HID] / deg[:, None]
    h = jnp.maximum(agg + xr_ref[...] + b1_ref[...][None, :], 0.0)
    dn = (((1,), (1,)), ((), ()))
    z = lax.dot_general(h, w2_ref[...], dn, preferred_element_type=_f32)
    s = jnp.sum(z * asrc_ref[...][None, :], axis=1)
    d = jnp.sum(z * adst_ref[...][None, :], axis=1)
    zpad = jnp.concatenate([z, jnp.zeros((R, W - HID), _f32)], axis=1)
    col = lax.broadcasted_iota(_i32, (R, W), 1)
    zaug = jnp.where(col == HID, 1.0, zpad)
    u_ref[0] = jnp.exp(s)[:, None] * zaug
    u_ref[1] = jnp.exp(0.2 * s)[:, None] * zaug
    s_ref[...] = s[:, None]
    d_ref[...] = d[:, None]
    z_ref[...] = z


_k2 = pl.pallas_call(
    _k2_body,
    grid=(N // R,),
    in_specs=[
        pl.BlockSpec((2, R, W), lambda i: (0, i, 0)),
        pl.BlockSpec((R, HID), lambda i: (i, 0)),
        pl.BlockSpec((HID,), lambda i: (0,)),
        pl.BlockSpec((C, HID), lambda i: (0, 0)),
        pl.BlockSpec((C,), lambda i: (0,)),
        pl.BlockSpec((C,), lambda i: (0,)),
    ],
    out_specs=[
        pl.BlockSpec((2, R, W), lambda i: (0, i, 0)),
        pl.BlockSpec((R, 1), lambda i: (i, 0)),
        pl.BlockSpec((R, 1), lambda i: (i, 0)),
        pl.BlockSpec((R, C), lambda i: (i, 0)),
    ],
    out_shape=[
        jax.ShapeDtypeStruct((2, NOFF, W), _f32),
        jax.ShapeDtypeStruct((N, 1), _f32),
        jax.ShapeDtypeStruct((N, 1), _f32),
        jax.ShapeDtypeStruct((N, C), _f32),
    ],
)


_RPT_B = 2 * NOFF // 16


@functools.partial(
    pl.kernel,
    out_type=jax.ShapeDtypeStruct((2, 2 * NOFF, W), _f32),
    mesh=_mesh,
    scratch_types=[
        pltpu.VMEM_SHARED((2 * NOFF, W), _f32),
        pltpu.VMEM((NTAB,), _f32),
        pltpu.VMEM((NTAB,), _f32),
        tuple(pltpu.VMEM((CHB,), _i32) for _ in range(4)),
        tuple(pltpu.VMEM((CHB,), _i32) for _ in range(4)),
        tuple(pltpu.VMEM((CHB,), _i32) for _ in range(4)),
        tuple(pltpu.VMEM((CHB,), _i32) for _ in range(4)),
        tuple(pltpu.VMEM((CHB, W), _f32) for _ in range(2)),
        tuple(pltpu.SemaphoreType.DMA for _ in range(2)),
        pltpu.SemaphoreType.DMA,
        tuple(pltpu.SemaphoreType.DMA for _ in range(2)),
    ],
    compiler_params=pltpu.CompilerParams(use_tc_tiling_on_sc=False, needs_layout_passes=False),
)
def _passB(u_hbm, s_hbm, d_hbm, src_hbm, dst_hbm, zeros_hbm, out_hbm,
           acc, stab, dtab, sbufs, dbufs, gbufs, wbufs, rowss,
           semi, semg, sems):
    cid = lax.axis_index("c")
    sid = lax.axis_index("s")
    wid = sid * 2 + cid
    pltpu.sync_copy(zeros_hbm, acc.at[pl.ds(sid * _RPT_B, _RPT_B)])
    pltpu.sync_copy(s_hbm, stab.at[pl.ds(0, N)])
    pltpu.sync_copy(d_hbm, dtab.at[pl.ds(0, N)])
    plsc.subcore_barrier()

    def issue_idx(j, ib, pb):
        base = (wid + j * NT) * CHB
        pltpu.async_copy(src_hbm.at[pl.ds(base, CHB)], sbufs[ib], semi[pb])
        pltpu.async_copy(dst_hbm.at[pl.ds(base, CHB)], dbufs[ib], semi[pb])

    def wait_idx(ib, pb):
        pltpu.make_async_copy(src_hbm.at[pl.ds(0, CHB)], sbufs[ib],
                              semi[pb]).wait()
        pltpu.make_async_copy(dst_hbm.at[pl.ds(0, CHB)], dbufs[ib],
                              semi[pb]).wait()

    def wait_scatter(rb):
        pltpu.make_async_copy(rowss[rb], acc.at[wbufs[0]], sems[rb]).wait()

    nj = jnp.where(wid < NRB, NJB_HI, NJB_LO)
    issue_idx(0, 0, 0)

    def outer(i, carry):
        jj = i * 4
        for b in range(4):
            j = jj + b
            ib, pb, rb = b, b % 2, b % 2

            @pl.when(j < nj)
            def _():
                @pl.when(j + 1 < nj)
                def _():
                    issue_idx(j + 1, (b + 1) % 4, (b + 1) % 2)
                wait_idx(ib, pb)
                for q in range(CHB // 16):
                    sv = sbufs[ib][pl.ds(q * 16, 16)]
                    dv = dbufs[ib][pl.ds(q * 16, 16)]
                    s16 = plsc.load_gather(stab, [sv])
                    d16 = plsc.load_gather(dtab, [dv])
                    neg = (s16 + d16) <= 0.0
                    off = jnp.where(neg, jnp.full((16,), NOFF, _i32),
                                    jnp.zeros((16,), _i32))
                    gbufs[ib][pl.ds(q * 16, 16)] = sv + off
                    wbufs[ib][pl.ds(q * 16, 16)] = dv + off

                @pl.when(j >= 2)
                def _():
                    wait_scatter(rb)
                pltpu.async_copy(u_hbm.at[gbufs[ib]], rowss[rb],
                                 semg).wait()
                pltpu.async_copy(rowss[rb], acc.at[wbufs[ib]], sems[rb],
                                 add=True)
        return carry

    lax.fori_loop(0, SBB // 4, outer, 0)
    wait_scatter(0)
    wait_scatter(1)
    plsc.subcore_barrier()
    pltpu.sync_copy(acc.at[pl.ds(sid * _RPT_B, _RPT_B)],
                    out_hbm.at[cid, pl.ds(sid * _RPT_B, _RPT_B)])


def _k3_body(accB_ref, s_ref, d_ref, z_ref, b2_ref, out_ref):
    p = accB_ref[0, 0] + accB_ref[1, 0]
    q = accB_ref[0, 1] + accB_ref[1, 1]
    s = s_ref[...][:, 0]
    d = d_ref[...][:, 0]
    full = jnp.exp(d)[:, None] * p + jnp.exp(0.2 * d)[:, None] * q
    t = s + d
    w = jnp.where(t > 0.0, jnp.exp(t), jnp.exp(0.2 * t))
    numer = full[:, :C] + w[:, None] * z_ref[...]
    denom = full[:, HID] + w
    logits = numer / denom[:, None] + b2_ref[...][None, :]
    mx = jnp.max(logits, axis=1, keepdims=True)
    sh = logits - mx
    out_ref[...] = sh - jnp.log(jnp.sum(jnp.exp(sh), axis=1, keepdims=True))


_k3 = pl.pallas_call(
    _k3_body,
    grid=(N // R,),
    in_specs=[
        pl.BlockSpec((2, 2, R, W), lambda i: (0, 0, i, 0)),
        pl.BlockSpec((R, 1), lambda i: (i, 0)),
        pl.BlockSpec((R, 1), lambda i: (i, 0)),
        pl.BlockSpec((R, C), lambda i: (i, 0)),
        pl.BlockSpec((C,), lambda i: (0,)),
    ],
    out_specs=pl.BlockSpec((R, C), lambda i: (i, 0)),
    out_shape=jax.ShapeDtypeStruct((N, C), _f32),
)


def kernel(x, edge_index, Wl, Wr, b1, W2, a_src, a_dst, b2):
    zeros2 = jnp.zeros((_RPT_B, W), _f32)
    src = edge_index[0]
    dst = edge_index[1]
    xlaug, xr = _k1(x, Wl, Wr)
    accA = _passA(xlaug, src, dst, zeros2)
    u, s, d, z = _k2(accA, xr, b1, W2, a_src, a_dst)
    accB = _passB(u.reshape(2 * NOFF, W), s.reshape(N), d.reshape(N),
                  src, dst, zeros2)
    return _k3(accB.reshape(2, 2, NOFF, W), s, d, z, b2)
